# baseline (device time: 12163 ns/iter reference)
import jax
import jax.numpy as jnp
from jax import lax
from jax.experimental import pallas as pl
from jax.experimental.pallas import tpu as pltpu

M = 512
NG = 1024
NCOL = 512

FTOT = 160
LTOT = 192
F_OFFS = (0, 48, 96, 128)
F_SIZES = (48, 48, 32, 32)
L_OFFS = (0, 48, 96, 144)
L_SIZES = (48, 48, 48, 48)
NF = len(F_OFFS)
NL = len(L_OFFS)
YROWS = FTOT + LTOT


def kernel(x):
    def body(
        x_ref,
        out_ref,
        xv,
        y_send,
        y_recv,
        z_recv,
        copy_sems,
        y_send_sems,
        y_recv_sems,
        z_send_sems,
        z_recv_sems,
    ):
        my_x = lax.axis_index("x")
        my_y = lax.axis_index("y")
        my_z = lax.axis_index("z")
        oy = 1 - my_y
        oz = 1 - my_z
        y_peer = (my_x, oy, my_z)
        z_peer = (my_x, my_y, oz)

        f0 = my_z * FTOT
        of0 = oz * FTOT
        mycols = pl.ds(my_y * NCOL, NCOL)
        oycols = pl.ds(oy * NCOL, NCOL)

        c_mine = pltpu.make_async_copy(
            x_ref.at[0, pl.ds(f0, FTOT), :],
            xv.at[pl.ds(f0, FTOT)],
            copy_sems.at[0],
        )
        c_L = pltpu.make_async_copy(
            x_ref.at[0, pl.ds(2 * FTOT, LTOT), :],
            xv.at[pl.ds(2 * FTOT, LTOT)],
            copy_sems.at[1],
        )
        c_other = pltpu.make_async_copy(
            x_ref.at[0, pl.ds(of0, FTOT), :],
            xv.at[pl.ds(of0, FTOT)],
            copy_sems.at[2],
        )
        c_mine.start()
        c_L.start()
        c_other.start()

        barrier_sem = pltpu.get_barrier_semaphore()
        for peer in (y_peer, z_peer):
            pl.semaphore_signal(
                barrier_sem, inc=1,
                device_id=peer,
                device_id_type=pl.DeviceIdType.MESH,
            )
        pl.semaphore_wait(barrier_sem, 2)

        def start_y(buf_off, x_row, s, sem_idx):
            y_send[pl.ds(buf_off, s), :] = xv[pl.ds(x_row, s), oycols].astype(
                jnp.bfloat16
            )
            r = pltpu.make_async_remote_copy(
                src_ref=y_send.at[pl.ds(buf_off, s)],
                dst_ref=y_recv.at[pl.ds(buf_off, s)],
                send_sem=y_send_sems.at[sem_idx],
                recv_sem=y_recv_sems.at[sem_idx],
                device_id=y_peer,
                device_id_type=pl.DeviceIdType.MESH,
            )
            r.start()
            return r

        c_mine.wait()
        yF = [
            start_y(o, f0 + o, s, i)
            for i, (o, s) in enumerate(zip(F_OFFS, F_SIZES))
        ]
        c_L.wait()
        yL = [
            start_y(FTOT + o, 2 * FTOT + o, s, NF + i)
            for i, (o, s) in enumerate(zip(L_OFFS, L_SIZES))
        ]

        z_rdmas = []
        for i, (o, s) in enumerate(zip(F_OFFS, F_SIZES)):
            yF[i].wait_recv()
            r = pltpu.make_async_remote_copy(
                src_ref=y_recv.at[pl.ds(o, s)],
                dst_ref=z_recv.at[pl.ds(o, s)],
                send_sem=z_send_sems.at[i],
                recv_sem=z_recv_sems.at[i],
                device_id=z_peer,
                device_id_type=pl.DeviceIdType.MESH,
            )
            r.start()
            z_rdmas.append(r)
            own = xv[pl.ds(f0 + o, s), mycols].astype(jnp.bfloat16)
            out_ref[pl.ds(f0 + o, s), :] = own + y_recv[pl.ds(o, s), :]

        for i, (o, s) in enumerate(zip(L_OFFS, L_SIZES)):
            yL[i].wait_recv()
            own = xv[pl.ds(2 * FTOT + o, s), mycols].astype(jnp.bfloat16)
            out_ref[pl.ds(2 * FTOT + o, s), :] = own + y_recv[
                pl.ds(FTOT + o, s), :
            ]

        c_other.wait()
        for i, (o, s) in enumerate(zip(F_OFFS, F_SIZES)):
            z_rdmas[i].wait_recv()
            own = xv[pl.ds(of0 + o, s), mycols].astype(jnp.bfloat16)
            out_ref[pl.ds(of0 + o, s), :] = own + z_recv[pl.ds(o, s), :]

        for r in yF + yL + z_rdmas:
            r.wait_send()

    return pl.pallas_call(
        body,
        out_shape=jax.ShapeDtypeStruct((M, NCOL), jnp.bfloat16),
        in_specs=[pl.BlockSpec(memory_space=pl.ANY)],
        out_specs=pl.BlockSpec(memory_space=pltpu.VMEM),
        scratch_shapes=[
            pltpu.VMEM((M, NG), jnp.float32),
            pltpu.VMEM((YROWS, NCOL), jnp.bfloat16),
            pltpu.VMEM((YROWS, NCOL), jnp.bfloat16),
            pltpu.VMEM((FTOT, NCOL), jnp.bfloat16),
            pltpu.SemaphoreType.DMA((3,)),
            pltpu.SemaphoreType.DMA((NF + NL,)),
            pltpu.SemaphoreType.DMA((NF + NL,)),
            pltpu.SemaphoreType.DMA((NF,)),
            pltpu.SemaphoreType.DMA((NF,)),
        ],
        compiler_params=pltpu.CompilerParams(collective_id=0),
    )(x)


# device time: 11583 ns/iter; 1.0501x vs baseline; 1.0501x over previous
import jax
import jax.numpy as jnp
from jax import lax
from jax.experimental import pallas as pl
from jax.experimental.pallas import tpu as pltpu

M = 512
NCOL = 512

FTOT = 160
LTOT = 192
F_OFFS = (0, 48, 96, 128)
F_SIZES = (48, 48, 32, 32)
L_OFFS = (0, 48, 96, 144)
L_SIZES = (48, 48, 48, 48)
NF = len(F_OFFS)
NL = len(L_OFFS)
YROWS = FTOT + LTOT


def kernel(x):
    def body(
        x_ref,
        out_ref,
        y_send,
        y_recv,
        z_recv,
        y_send_sems,
        y_recv_sems,
        z_send_sems,
        z_recv_sems,
    ):
        my_x = lax.axis_index("x")
        my_y = lax.axis_index("y")
        my_z = lax.axis_index("z")
        oy = 1 - my_y
        oz = 1 - my_z
        y_peer = (my_x, oy, my_z)
        z_peer = (my_x, my_y, oz)

        barrier_sem = pltpu.get_barrier_semaphore()
        for peer in (y_peer, z_peer):
            pl.semaphore_signal(
                barrier_sem, inc=1,
                device_id=peer,
                device_id_type=pl.DeviceIdType.MESH,
            )
        pl.semaphore_wait(barrier_sem, 2)

        f0 = my_z * FTOT
        of0 = oz * FTOT
        mycols = pl.ds(my_y * NCOL, NCOL)

        def start_y(buf_off, x_row, size, sem_idx):
            y_send[pl.ds(buf_off, size), :] = x_ref[
                0, pl.ds(x_row, size), pl.ds(oy * NCOL, NCOL)
            ].astype(jnp.bfloat16)
            r = pltpu.make_async_remote_copy(
                src_ref=y_send.at[pl.ds(buf_off, size)],
                dst_ref=y_recv.at[pl.ds(buf_off, size)],
                send_sem=y_send_sems.at[sem_idx],
                recv_sem=y_recv_sems.at[sem_idx],
                device_id=y_peer,
                device_id_type=pl.DeviceIdType.MESH,
            )
            r.start()
            return r

        yF = [
            start_y(o, f0 + o, s, i)
            for i, (o, s) in enumerate(zip(F_OFFS, F_SIZES))
        ]
        yL = [
            start_y(FTOT + o, 2 * FTOT + o, s, NF + i)
            for i, (o, s) in enumerate(zip(L_OFFS, L_SIZES))
        ]

        z_rdmas = []
        for i, (o, s) in enumerate(zip(F_OFFS, F_SIZES)):
            yF[i].wait_recv()
            r = pltpu.make_async_remote_copy(
                src_ref=y_recv.at[pl.ds(o, s)],
                dst_ref=z_recv.at[pl.ds(o, s)],
                send_sem=z_send_sems.at[i],
                recv_sem=z_recv_sems.at[i],
                device_id=z_peer,
                device_id_type=pl.DeviceIdType.MESH,
            )
            r.start()
            z_rdmas.append(r)
            own = x_ref[0, pl.ds(f0 + o, s), mycols].astype(jnp.bfloat16)
            out_ref[pl.ds(f0 + o, s), :] = own + y_recv[pl.ds(o, s), :]

        for i, (o, s) in enumerate(zip(F_OFFS, F_SIZES)):
            z_rdmas[i].wait_recv()
            own = x_ref[0, pl.ds(of0 + o, s), mycols].astype(jnp.bfloat16)
            out_ref[pl.ds(of0 + o, s), :] = own + z_recv[pl.ds(o, s), :]

        for i, (o, s) in enumerate(zip(L_OFFS, L_SIZES)):
            yL[i].wait_recv()
            own = x_ref[0, pl.ds(2 * FTOT + o, s), mycols].astype(jnp.bfloat16)
            out_ref[pl.ds(2 * FTOT + o, s), :] = own + y_recv[
                pl.ds(FTOT + o, s), :
            ]

        for r in yF + yL + z_rdmas:
            r.wait_send()

    return pl.pallas_call(
        body,
        out_shape=jax.ShapeDtypeStruct((M, NCOL), jnp.bfloat16),
        in_specs=[pl.BlockSpec(memory_space=pltpu.VMEM)],
        out_specs=pl.BlockSpec(memory_space=pltpu.VMEM),
        scratch_shapes=[
            pltpu.VMEM((YROWS, NCOL), jnp.bfloat16),
            pltpu.VMEM((YROWS, NCOL), jnp.bfloat16),
            pltpu.VMEM((FTOT, NCOL), jnp.bfloat16),
            pltpu.SemaphoreType.DMA((NF + NL,)),
            pltpu.SemaphoreType.DMA((NF + NL,)),
            pltpu.SemaphoreType.DMA((NF,)),
            pltpu.SemaphoreType.DMA((NF,)),
        ],
        compiler_params=pltpu.CompilerParams(collective_id=0),
    )(x)


# device time: 11473 ns/iter; 1.0601x vs baseline; 1.0096x over previous
import jax
import jax.numpy as jnp
from jax import lax
from jax.experimental import pallas as pl
from jax.experimental.pallas import tpu as pltpu

M = 512
NCOL = 512

FTOT = 176
LTOT = 160
F_OFFS = (0, 48, 96, 144)
F_SIZES = (48, 48, 48, 32)
L_OFFS = (0, 48, 96, 128)
L_SIZES = (48, 48, 32, 32)
NF = len(F_OFFS)
NL = len(L_OFFS)
YROWS = FTOT + LTOT


def kernel(x):
    def body(
        x_ref,
        out_ref,
        y_send,
        y_recv,
        z_recv,
        y_send_sems,
        y_recv_sems,
        z_send_sems,
        z_recv_sems,
    ):
        my_x = lax.axis_index("x")
        my_y = lax.axis_index("y")
        my_z = lax.axis_index("z")
        oy = 1 - my_y
        oz = 1 - my_z
        y_peer = (my_x, oy, my_z)
        z_peer = (my_x, my_y, oz)

        barrier_sem = pltpu.get_barrier_semaphore()
        for peer in (y_peer, z_peer):
            pl.semaphore_signal(
                barrier_sem, inc=1,
                device_id=peer,
                device_id_type=pl.DeviceIdType.MESH,
            )
        pl.semaphore_wait(barrier_sem, 2)

        f0 = my_z * FTOT
        of0 = oz * FTOT
        mycols = pl.ds(my_y * NCOL, NCOL)

        def start_y(buf_off, x_row, size, sem_idx):
            y_send[pl.ds(buf_off, size), :] = x_ref[
                0, pl.ds(x_row, size), pl.ds(oy * NCOL, NCOL)
            ].astype(jnp.bfloat16)
            r = pltpu.make_async_remote_copy(
                src_ref=y_send.at[pl.ds(buf_off, size)],
                dst_ref=y_recv.at[pl.ds(buf_off, size)],
                send_sem=y_send_sems.at[sem_idx],
                recv_sem=y_recv_sems.at[sem_idx],
                device_id=y_peer,
                device_id_type=pl.DeviceIdType.MESH,
            )
            r.start()
            return r

        yF = [
            start_y(o, f0 + o, s, i)
            for i, (o, s) in enumerate(zip(F_OFFS, F_SIZES))
        ]
        yL = [
            start_y(FTOT + o, 2 * FTOT + o, s, NF + i)
            for i, (o, s) in enumerate(zip(L_OFFS, L_SIZES))
        ]

        z_rdmas = []
        for i, (o, s) in enumerate(zip(F_OFFS, F_SIZES)):
            yF[i].wait_recv()
            r = pltpu.make_async_remote_copy(
                src_ref=y_recv.at[pl.ds(o, s)],
                dst_ref=z_recv.at[pl.ds(o, s)],
                send_sem=z_send_sems.at[i],
                recv_sem=z_recv_sems.at[i],
                device_id=z_peer,
                device_id_type=pl.DeviceIdType.MESH,
            )
            r.start()
            z_rdmas.append(r)
            own = x_ref[0, pl.ds(f0 + o, s), mycols].astype(jnp.bfloat16)
            out_ref[pl.ds(f0 + o, s), :] = own + y_recv[pl.ds(o, s), :]

        for i, (o, s) in enumerate(zip(F_OFFS, F_SIZES)):
            z_rdmas[i].wait_recv()
            own = x_ref[0, pl.ds(of0 + o, s), mycols].astype(jnp.bfloat16)
            out_ref[pl.ds(of0 + o, s), :] = own + z_recv[pl.ds(o, s), :]

        for i, (o, s) in enumerate(zip(L_OFFS, L_SIZES)):
            yL[i].wait_recv()
            own = x_ref[0, pl.ds(2 * FTOT + o, s), mycols].astype(jnp.bfloat16)
            out_ref[pl.ds(2 * FTOT + o, s), :] = own + y_recv[
                pl.ds(FTOT + o, s), :
            ]

        for r in yF + yL + z_rdmas:
            r.wait_send()

    return pl.pallas_call(
        body,
        out_shape=jax.ShapeDtypeStruct((M, NCOL), jnp.bfloat16),
        in_specs=[pl.BlockSpec(memory_space=pltpu.VMEM)],
        out_specs=pl.BlockSpec(memory_space=pltpu.VMEM),
        scratch_shapes=[
            pltpu.VMEM((YROWS, NCOL), jnp.bfloat16),
            pltpu.VMEM((YROWS, NCOL), jnp.bfloat16),
            pltpu.VMEM((FTOT, NCOL), jnp.bfloat16),
            pltpu.SemaphoreType.DMA((NF + NL,)),
            pltpu.SemaphoreType.DMA((NF + NL,)),
            pltpu.SemaphoreType.DMA((NF,)),
            pltpu.SemaphoreType.DMA((NF,)),
        ],
        compiler_params=pltpu.CompilerParams(collective_id=0),
    )(x)
